# baseline (speedup 1.0000x reference)
import jax
import jax.numpy as jnp
from jax.experimental import pallas as pl
from jax.experimental.pallas import tpu as pltpu

BT = 512
BF = 1024
NJ = 4


def _ffn_kernel(x_ref, w1_ref, b1_ref, w2_ref, b2_ref, o_ref, h_ref):
    j = pl.program_id(1)

    @pl.when(j < NJ)
    def _():
        h = jnp.dot(x_ref[...], w1_ref[...], preferred_element_type=jnp.float32)
        h = h + b1_ref[...]
        h = h * 0.5 * (1.0 + jax.lax.erf(h * 0.7071067811865476))
        h_ref[j % 2] = h

    @pl.when(j == 1)
    def _():
        o_ref[...] = jnp.dot(h_ref[0], w2_ref[...],
                             preferred_element_type=jnp.float32) + b2_ref[...]

    @pl.when(j > 1)
    def _():
        o_ref[...] = o_ref[...] + jnp.dot(h_ref[(j - 1) % 2], w2_ref[...],
                                          preferred_element_type=jnp.float32)


def kernel(inputs, W1, b1, W2, b2, input_split):
    del input_split
    tokens, d_model = inputs.shape
    d_ff = W1.shape[1]
    nj = d_ff // BF
    b1_2d = b1.reshape(1, d_ff)
    b2_2d = b2.reshape(1, d_model)
    return pl.pallas_call(
        _ffn_kernel,
        grid=(tokens // BT, nj + 1),
        in_specs=[
            pl.BlockSpec((BT, d_model), lambda i, j: (i, 0)),
            pl.BlockSpec((d_model, BF), lambda i, j: (0, jnp.minimum(j, NJ - 1))),
            pl.BlockSpec((1, BF), lambda i, j: (0, jnp.minimum(j, NJ - 1))),
            pl.BlockSpec((BF, d_model), lambda i, j: (jnp.maximum(j, 1) - 1, 0)),
            pl.BlockSpec((1, d_model), lambda i, j: (0, 0)),
        ],
        out_specs=pl.BlockSpec((BT, d_model), lambda i, j: (i, 0)),
        out_shape=jax.ShapeDtypeStruct((tokens, d_model), jnp.float32),
        scratch_shapes=[pltpu.VMEM((2, BT, BF), jnp.float32)],
        compiler_params=pltpu.CompilerParams(
            dimension_semantics=("parallel", "arbitrary"),
            vmem_limit_bytes=100 * 1024 * 1024),
    )(inputs, W1, b1_2d, W2, b2_2d)


# speedup vs baseline: 1.0357x; 1.0357x over previous
import jax
import jax.numpy as jnp
from jax.experimental import pallas as pl
from jax.experimental.pallas import tpu as pltpu

BT = 512
BF = 1024
NJ = 4
NI = 8


def _ffn_kernel(x_ref, w1_ref, b1_ref, w2_ref, b2_ref, o_ref, h_ref):
    s = pl.program_id(0)

    contrib = jnp.dot(h_ref[(s - 1) % 2], w2_ref[...],
                      preferred_element_type=jnp.float32)

    h = jnp.dot(x_ref[...], w1_ref[...], preferred_element_type=jnp.float32)
    h = h + b1_ref[...]
    h = h * 0.5 * (1.0 + jax.lax.erf(h * 0.7071067811865476))
    h_ref[s % 2] = h

    @pl.when(s % NJ == 1)
    def _():
        o_ref[...] = contrib + b2_ref[...]

    @pl.when((s != 0) & (s % NJ != 1))
    def _():
        o_ref[...] = o_ref[...] + contrib


def kernel(inputs, W1, b1, W2, b2, input_split):
    del input_split
    tokens, d_model = inputs.shape
    d_ff = W1.shape[1]
    b1_2d = b1.reshape(1, d_ff)
    b2_2d = b2.reshape(1, d_model)
    return pl.pallas_call(
        _ffn_kernel,
        grid=(NI * NJ + 1,),
        in_specs=[
            pl.BlockSpec((BT, d_model),
                         lambda s: (jnp.minimum(s // NJ, NI - 1), 0)),
            pl.BlockSpec((d_model, BF), lambda s: (0, s % NJ)),
            pl.BlockSpec((1, BF), lambda s: (0, s % NJ)),
            pl.BlockSpec((BF, d_model), lambda s: ((s - 1) % NJ, 0)),
            pl.BlockSpec((1, d_model), lambda s: (0, 0)),
        ],
        out_specs=pl.BlockSpec(
            (BT, d_model),
            lambda s: (jnp.maximum(s - 1, 0) // NJ, 0)),
        out_shape=jax.ShapeDtypeStruct((tokens, d_model), jnp.float32),
        scratch_shapes=[pltpu.VMEM((2, BT, BF), jnp.float32)],
        compiler_params=pltpu.CompilerParams(
            dimension_semantics=("arbitrary",),
            vmem_limit_bytes=100 * 1024 * 1024),
    )(inputs, W1, b1_2d, W2, b2_2d)


# speedup vs baseline: 1.1848x; 1.1440x over previous
import jax
import jax.numpy as jnp
from jax.experimental import pallas as pl
from jax.experimental.pallas import tpu as pltpu

BT = 1024
BF = 512


def _ffn_kernel(x_ref, w1_ref, b1_ref, w2_ref, b2_ref, o_ref):
    j = pl.program_id(1)
    h = jnp.dot(x_ref[...], w1_ref[...], preferred_element_type=jnp.float32)
    h = h + b1_ref[...]
    h = h * 0.5 * (1.0 + jax.lax.erf(h * 0.7071067811865476))
    contrib = jnp.dot(h, w2_ref[...], preferred_element_type=jnp.float32)

    @pl.when(j == 0)
    def _():
        o_ref[...] = contrib + b2_ref[...]

    @pl.when(j != 0)
    def _():
        o_ref[...] = o_ref[...] + contrib


def kernel(inputs, W1, b1, W2, b2, input_split):
    del input_split
    tokens, d_model = inputs.shape
    d_ff = W1.shape[1]
    b1_2d = b1.reshape(1, d_ff)
    b2_2d = b2.reshape(1, d_model)
    return pl.pallas_call(
        _ffn_kernel,
        grid=(tokens // BT, d_ff // BF),
        in_specs=[
            pl.BlockSpec((BT, d_model), lambda i, j: (i, 0)),
            pl.BlockSpec((d_model, BF), lambda i, j: (0, j)),
            pl.BlockSpec((1, BF), lambda i, j: (0, j)),
            pl.BlockSpec((BF, d_model), lambda i, j: (j, 0)),
            pl.BlockSpec((1, d_model), lambda i, j: (0, 0)),
        ],
        out_specs=pl.BlockSpec((BT, d_model), lambda i, j: (i, 0)),
        out_shape=jax.ShapeDtypeStruct((tokens, d_model), jnp.float32),
        compiler_params=pltpu.CompilerParams(
            dimension_semantics=("parallel", "arbitrary"),
            vmem_limit_bytes=100 * 1024 * 1024),
    )(inputs, W1, b1_2d, W2, b2_2d)
